# Initial kernel scaffold; baseline (speedup 1.0000x reference)
import functools

import jax
import jax.numpy as jnp
import numpy as np
from jax import lax
from jax.experimental import pallas as pl
from jax.experimental.pallas import tpu as pltpu
from jax.experimental.pallas import tpu_sc as plsc

HID = 128
NH = 4
HD = HID // NH
K = 10
N = 4096
Q = 2048
EDGES = 65536
NE = 4096

NC = 2
NS = 16

_SQRT_HD = np.sqrt(np.float32(HD)).astype(np.float32)


_KNN_BLK = 256


def _knn_body(xb_ref, xf_ref, out_ref):
    xb = xb_ref[...]
    xf = xf_ref[...]
    sq_i = jnp.sum(xb * xb, axis=1, keepdims=True)
    sq_j = jnp.sum(xf * xf, axis=1)
    s = lax.dot_general(xb, xf, (((1,), (1,)), ((), ())),
                        preferred_element_type=jnp.float32)
    d2 = sq_i + sq_j[None, :] - 2.0 * s
    dist = jnp.sqrt(jnp.clip(d2, 0.0, None))
    col = lax.broadcasted_iota(jnp.int32, dist.shape, 1)
    cur = dist
    big_i = jnp.int32(2**30)
    cols = []
    for _ in range(K):
        m = jnp.min(cur, axis=1, keepdims=True)
        sel = jnp.where(cur == m, col, big_i)
        idx = jnp.min(sel, axis=1)
        cols.append(idx)
        cur = jnp.where(col == idx[:, None], jnp.inf, cur)
    for _ in range(16 - K):
        cols.append(cols[-1])
    out_ref[...] = jnp.stack(cols, axis=1)


def _tc_knn(x):
    return pl.pallas_call(
        _knn_body,
        grid=(N // _KNN_BLK,),
        in_specs=[
            pl.BlockSpec((_KNN_BLK, HID), lambda i: (i, 0)),
            pl.BlockSpec((N, HID), lambda i: (0, 0)),
        ],
        out_specs=pl.BlockSpec((_KNN_BLK, 16), lambda i: (i, 0)),
        out_shape=jax.ShapeDtypeStruct((N, 16), jnp.int32),
    )(x, x)


def _mha_blocks(q, k_s, v_s, out_w, out_b):
    heads = []
    for h in range(NH):
        qh = q[:, h * HD:(h + 1) * HD]
        kh = k_s[:, h * HD:(h + 1) * HD]
        vh = v_s[:, h * HD:(h + 1) * HD]
        s = lax.dot_general(qh, kh, (((1,), (1,)), ((), ())),
                            preferred_element_type=jnp.float32) / _SQRT_HD
        m = jnp.max(s, axis=1, keepdims=True)
        e = jnp.exp(s - m)
        p = e / jnp.sum(e, axis=1, keepdims=True)
        oh = lax.dot_general(p, vh, (((1,), (0,)), ((), ())),
                             preferred_element_type=jnp.float32)
        heads.append(oh)
    o = jnp.concatenate(heads, axis=1)
    return lax.dot_general(o, out_w, (((1,), (1,)), ((), ())),
                           preferred_element_type=jnp.float32) + out_b


def _mha1_body(q_in_ref, kv_ref, in_w_ref, in_b_ref, out_w_ref, out_b_ref,
               o_ref, k_s, v_s):
    in_w = in_w_ref[...]
    in_b = in_b_ref[...]

    @pl.when(pl.program_id(0) == 0)
    def _():
        kv = kv_ref[...]
        k_s[...] = lax.dot_general(kv, in_w[HID:2 * HID, :],
                                   (((1,), (1,)), ((), ())),
                                   preferred_element_type=jnp.float32) \
            + in_b[:, HID:2 * HID]
        v_s[...] = lax.dot_general(kv, in_w[2 * HID:, :],
                                   (((1,), (1,)), ((), ())),
                                   preferred_element_type=jnp.float32) \
            + in_b[:, 2 * HID:]

    q = lax.dot_general(q_in_ref[...], in_w[:HID, :], (((1,), (1,)), ((), ())),
                        preferred_element_type=jnp.float32) + in_b[:, :HID]
    o_ref[...] = _mha_blocks(q, k_s[...], v_s[...], out_w_ref[...],
                             out_b_ref[...])


_MHA_BLK = 512


def _tc_mha(q_in, kv, in_w, in_b2, out_w, out_b2):
    lq = q_in.shape[0]
    lk = kv.shape[0]
    return pl.pallas_call(
        _mha1_body,
        grid=(lq // _MHA_BLK,),
        in_specs=[
            pl.BlockSpec((_MHA_BLK, HID), lambda i: (i, 0)),
            pl.BlockSpec((lk, HID), lambda i: (0, 0)),
            pl.BlockSpec((3 * HID, HID), lambda i: (0, 0)),
            pl.BlockSpec((1, 3 * HID), lambda i: (0, 0)),
            pl.BlockSpec((HID, HID), lambda i: (0, 0)),
            pl.BlockSpec((1, HID), lambda i: (0, 0)),
        ],
        out_specs=pl.BlockSpec((_MHA_BLK, HID), lambda i: (i, 0)),
        out_shape=jax.ShapeDtypeStruct((lq, HID), jnp.float32),
        scratch_shapes=[
            pltpu.VMEM((lk, HID), jnp.float32),
            pltpu.VMEM((lk, HID), jnp.float32),
        ],
    )(q_in, kv, in_w, in_b2, out_w, out_b2)


_RED_BLK = 512


def _reduce_mm_body(g_ref, x_ref, w_ref, o_ref):
    inter = jnp.sum(g_ref[...], axis=1)
    x2 = x_ref[...] + inter
    o_ref[...] = lax.dot_general(x2, w_ref[...], (((1,), (1,)), ((), ())),
                                 preferred_element_type=jnp.float32)


def _tc_reduce_mm(g3, x, w_hg):
    return pl.pallas_call(
        _reduce_mm_body,
        grid=(N // _RED_BLK,),
        in_specs=[
            pl.BlockSpec((_RED_BLK, K, HID), lambda i: (i, 0, 0)),
            pl.BlockSpec((_RED_BLK, HID), lambda i: (i, 0)),
            pl.BlockSpec((HID, HID), lambda i: (0, 0)),
        ],
        out_specs=pl.BlockSpec((_RED_BLK, HID), lambda i: (i, 0)),
        out_shape=jax.ShapeDtypeStruct((N, HID), jnp.float32),
    )(g3, x, w_hg)


def _combine_ef_body(e0_ref, e1_ref, bd0_ref, bd1_ref, o_ref):
    bd = bd0_ref[0][:, :1] + bd1_ref[0][:, :1]
    binv = jnp.where(bd > 0, 1.0 / bd, 0.0)
    o_ref[...] = (e0_ref[0] + e1_ref[0]) * binv


def _tc_combine_ef(ef_part, bd_part):
    blk = 512
    return pl.pallas_call(
        _combine_ef_body,
        grid=(NE // blk,),
        in_specs=[
            pl.BlockSpec((1, blk, HID), lambda i: (0, i, 0)),
            pl.BlockSpec((1, blk, HID), lambda i: (1, i, 0)),
            pl.BlockSpec((1, blk, 16), lambda i: (0, i, 0)),
            pl.BlockSpec((1, blk, 16), lambda i: (1, i, 0)),
        ],
        out_specs=pl.BlockSpec((blk, HID), lambda i: (i, 0)),
        out_shape=jax.ShapeDtypeStruct((NE, HID), jnp.float32),
    )(ef_part, ef_part, bd_part, bd_part)


def _final_body(op_ref, dp_ref, bhg_ref, gam_ref, bet_ref, m_ref,
                in_w_ref, in_b_ref, out_w_ref, out_b_ref, o_ref, k_s, v_s):
    in_w = in_w_ref[...]
    in_b = in_b_ref[...]

    @pl.when(pl.program_id(0) == 0)
    def _():
        d = dp_ref[0, :, :1] + dp_ref[1, :, :1]
        dinv = jnp.where(d > 0, 1.0 / d, 0.0)
        out = (op_ref[0] + op_ref[1]) * dinv + bhg_ref[...]
        mu = jnp.mean(out, axis=0, keepdims=True)
        var = jnp.mean((out - mu) ** 2, axis=0, keepdims=True)
        out = gam_ref[...] * (out - mu) / jnp.sqrt(var + 1e-5) + bet_ref[...]
        out = jnp.where(out > 0, out, jnp.exp(jnp.minimum(out, 0.0)) - 1.0)
        k_s[...] = lax.dot_general(out, in_w[HID:2 * HID, :],
                                   (((1,), (1,)), ((), ())),
                                   preferred_element_type=jnp.float32) \
            + in_b[:, HID:2 * HID]
        v_s[...] = lax.dot_general(out, in_w[2 * HID:, :],
                                   (((1,), (1,)), ((), ())),
                                   preferred_element_type=jnp.float32) \
            + in_b[:, 2 * HID:]

    q = lax.dot_general(m_ref[...], in_w[:HID, :], (((1,), (1,)), ((), ())),
                        preferred_element_type=jnp.float32) + in_b[:, :HID]
    o_ref[...] = _mha_blocks(q, k_s[...], v_s[...], out_w_ref[...],
                             out_b_ref[...])


def _tc_final(out_part, d_part, b_hg2, gamma2, beta2, m_embs,
              in_w, in_b2, out_w, out_b2):
    return pl.pallas_call(
        _final_body,
        grid=(Q // _MHA_BLK,),
        in_specs=[
            pl.BlockSpec((2, N, HID), lambda i: (0, 0, 0)),
            pl.BlockSpec((2, N, 16), lambda i: (0, 0, 0)),
            pl.BlockSpec((1, HID), lambda i: (0, 0)),
            pl.BlockSpec((1, HID), lambda i: (0, 0)),
            pl.BlockSpec((1, HID), lambda i: (0, 0)),
            pl.BlockSpec((_MHA_BLK, HID), lambda i: (i, 0)),
            pl.BlockSpec((3 * HID, HID), lambda i: (0, 0)),
            pl.BlockSpec((1, 3 * HID), lambda i: (0, 0)),
            pl.BlockSpec((HID, HID), lambda i: (0, 0)),
            pl.BlockSpec((1, HID), lambda i: (0, 0)),
        ],
        out_specs=pl.BlockSpec((_MHA_BLK, HID), lambda i: (i, 0)),
        out_shape=jax.ShapeDtypeStruct((Q, HID), jnp.float32),
        scratch_shapes=[
            pltpu.VMEM((N, HID), jnp.float32),
            pltpu.VMEM((N, HID), jnp.float32),
        ],
    )(out_part, d_part, b_hg2, gamma2, beta2, m_embs, in_w, in_b2, out_w,
      out_b2)


_SC_MESH = plsc.VectorSubcoreMesh(core_axis_name="c", subcore_axis_name="s")
_CHUNK = 512
_EDGE_PER_TILE = EDGES // (NC * NS)
_EDGE_CHUNKS = _EDGE_PER_TILE // _CHUNK
_STRIPE = N // NS


def _sc_counts(nodes, edges, zeros16, ones16):
    @functools.partial(
        pl.kernel, mesh=_SC_MESH,
        out_type=[
            jax.ShapeDtypeStruct((NC, N, 16), jnp.float32),
            jax.ShapeDtypeStruct((NC, NE, 16), jnp.float32),
        ],
        scratch_types=[
            pltpu.VMEM((_CHUNK,), jnp.int32),
            pltpu.VMEM((_CHUNK, 16), jnp.float32),
            pltpu.VMEM_SHARED((N, 16), jnp.float32),
            pltpu.VMEM_SHARED((NE, 16), jnp.float32),
        ],
    )
    def k(nodes_hbm, edges_hbm, z_hbm, ones_hbm, d_out, bd_out,
          idx_v, ones_v, d_sh, bd_sh):
        c = lax.axis_index("c")
        s = lax.axis_index("s")
        stripe = s * _STRIPE
        pltpu.sync_copy(z_hbm.at[pl.ds(stripe, _STRIPE)],
                        d_sh.at[pl.ds(stripe, _STRIPE)])
        pltpu.sync_copy(z_hbm.at[pl.ds(stripe, _STRIPE)],
                        bd_sh.at[pl.ds(stripe, _STRIPE)])
        pltpu.sync_copy(ones_hbm, ones_v)
        plsc.subcore_barrier()
        base = c * (EDGES // NC) + s * _EDGE_PER_TILE
        for j in range(_EDGE_CHUNKS):
            pltpu.sync_copy(nodes_hbm.at[pl.ds(base + j * _CHUNK, _CHUNK)],
                            idx_v)
            pltpu.sync_copy(ones_v, d_sh.at[idx_v], add=True)
            pltpu.sync_copy(edges_hbm.at[pl.ds(base + j * _CHUNK, _CHUNK)],
                            idx_v)
            pltpu.sync_copy(ones_v, bd_sh.at[idx_v], add=True)
        plsc.subcore_barrier()
        pltpu.sync_copy(d_sh.at[pl.ds(stripe, _STRIPE)],
                        d_out.at[c].at[pl.ds(stripe, _STRIPE)])
        pltpu.sync_copy(bd_sh.at[pl.ds(stripe, _STRIPE)],
                        bd_out.at[c].at[pl.ds(stripe, _STRIPE)])

    return k(nodes, edges, zeros16, ones16)


def _sc_gather(table, idx):
    b = idx.shape[0]
    per_w = b // (NC * NS)
    chunk = 256
    nch = per_w // chunk

    @functools.partial(
        pl.kernel, mesh=_SC_MESH,
        out_type=jax.ShapeDtypeStruct((b, HID), jnp.float32),
        scratch_types=[
            pltpu.VMEM((chunk,), jnp.int32),
            pltpu.VMEM((chunk, HID), jnp.float32),
            pltpu.SemaphoreType.DMA,
        ],
    )
    def k(tab_hbm, idx_hbm, o_hbm, idx_v, rows_v, sem):
        c = lax.axis_index("c")
        s = lax.axis_index("s")
        wid = s * NC + c
        base = wid * per_w
        for j in range(nch):
            off = base + j * chunk
            pltpu.sync_copy(idx_hbm.at[pl.ds(off, chunk)], idx_v)
            pltpu.async_copy(tab_hbm.at[idx_v], rows_v, sem).wait()
            pltpu.sync_copy(rows_v, o_hbm.at[pl.ds(off, chunk)])

    return k(table, idx)


def _sc_segsum(values, gather_idx, scatter_idx, zeros128):
    @functools.partial(
        pl.kernel, mesh=_SC_MESH,
        out_type=jax.ShapeDtypeStruct((NC, N, HID), jnp.float32),
        scratch_types=[
            pltpu.VMEM((_CHUNK,), jnp.int32),
            pltpu.VMEM((_CHUNK,), jnp.int32),
            pltpu.VMEM((_CHUNK, HID), jnp.float32),
            pltpu.VMEM_SHARED((N, HID), jnp.float32),
            pltpu.SemaphoreType.DMA,
        ],
    )
    def k(val_hbm, gi_hbm, si_hbm, z_hbm, o_hbm, gi_v, si_v, rows_v, acc_sh,
          sem):
        c = lax.axis_index("c")
        s = lax.axis_index("s")
        stripe = s * _STRIPE
        pltpu.sync_copy(z_hbm.at[pl.ds(stripe, _STRIPE)],
                        acc_sh.at[pl.ds(stripe, _STRIPE)])
        plsc.subcore_barrier()
        base = c * (EDGES // NC) + s * _EDGE_PER_TILE
        for j in range(_EDGE_CHUNKS):
            off = base + j * _CHUNK
            pltpu.sync_copy(gi_hbm.at[pl.ds(off, _CHUNK)], gi_v)
            pltpu.sync_copy(si_hbm.at[pl.ds(off, _CHUNK)], si_v)
            pltpu.async_copy(val_hbm.at[gi_v], rows_v, sem).wait()
            pltpu.sync_copy(rows_v, acc_sh.at[si_v], add=True)
        plsc.subcore_barrier()
        pltpu.sync_copy(acc_sh.at[pl.ds(stripe, _STRIPE)],
                        o_hbm.at[c].at[pl.ds(stripe, _STRIPE)])

    return k(values, gather_idx, scatter_idx, zeros128)


def kernel(embs1, embs2, m_embs, edge_index, W_hg, b_hg, bn_gamma, bn_beta,
           attn_in_w, attn_in_b, attn_out_w, attn_out_b,
           mha_in_w, mha_in_b, mha_out_w, mha_out_b):
    x = jnp.concatenate([embs1, embs2], axis=0)
    nodes = edge_index[0]
    edges = edge_index[1]
    he = jax.random.normal(jax.random.key(1), (N, HID), dtype=jnp.float32)

    zeros16 = jnp.zeros((N, 16), jnp.float32)
    ones16 = jnp.ones((_CHUNK, 16), jnp.float32)
    zeros128 = jnp.zeros((N, HID), jnp.float32)

    d_part, bd_part = _sc_counts(nodes, edges, zeros16, ones16)

    nearest = _tc_knn(x)
    he_out = _tc_mha(he, x, attn_in_w, attn_in_b.reshape(1, -1),
                     attn_out_w, attn_out_b.reshape(1, -1))

    idx_flat = nearest[:, :K].reshape(-1)
    g = _sc_gather(he_out, idx_flat)
    xw = _tc_reduce_mm(g.reshape(N, K, HID), x, W_hg)

    ef_part = _sc_segsum(xw, nodes, edges, zeros128)
    ef = _tc_combine_ef(ef_part, bd_part)
    out_part = _sc_segsum(ef, edges, nodes, zeros128)

    return _tc_final(out_part, d_part, b_hg.reshape(1, -1),
                     bn_gamma.reshape(1, -1), bn_beta.reshape(1, -1),
                     m_embs, mha_in_w, mha_in_b.reshape(1, -1),
                     mha_out_w, mha_out_b.reshape(1, -1))


# speedup vs baseline: 5.6634x; 5.6634x over previous
import functools

import jax
import jax.numpy as jnp
import numpy as np
from jax import lax
from jax.experimental import pallas as pl
from jax.experimental.pallas import tpu as pltpu
from jax.experimental.pallas import tpu_sc as plsc

HID = 128
NH = 4
HD = HID // NH
K = 10
N = 4096
Q = 2048
EDGES = 65536
NE = 4096

NC = 2
NS = 16

_SQRT_HD = np.sqrt(np.float32(HD)).astype(np.float32)


_KNN_BLK = 256


def _knn_body(xb_ref, xf_ref, out_ref):
    xb = xb_ref[...]
    xf = xf_ref[...]
    sq_i = jnp.sum(xb * xb, axis=1, keepdims=True)
    sq_j = jnp.sum(xf * xf, axis=1)
    s = lax.dot_general(xb, xf, (((1,), (1,)), ((), ())),
                        preferred_element_type=jnp.float32)
    d2 = sq_i + sq_j[None, :] - 2.0 * s
    dist = jnp.sqrt(jnp.clip(d2, 0.0, None))
    col = lax.broadcasted_iota(jnp.int32, dist.shape, 1)
    cur = dist
    big_i = jnp.int32(2**30)
    cols = []
    for _ in range(K):
        m = jnp.min(cur, axis=1, keepdims=True)
        sel = jnp.where(cur == m, col, big_i)
        idx = jnp.min(sel, axis=1)
        cols.append(idx)
        cur = jnp.where(col == idx[:, None], jnp.inf, cur)
    for _ in range(16 - K):
        cols.append(cols[-1])
    out_ref[...] = jnp.stack(cols, axis=1)


def _tc_knn(x):
    return pl.pallas_call(
        _knn_body,
        grid=(N // _KNN_BLK,),
        in_specs=[
            pl.BlockSpec((_KNN_BLK, HID), lambda i: (i, 0)),
            pl.BlockSpec((N, HID), lambda i: (0, 0)),
        ],
        out_specs=pl.BlockSpec((_KNN_BLK, 16), lambda i: (i, 0)),
        out_shape=jax.ShapeDtypeStruct((N, 16), jnp.int32),
    )(x, x)


def _mha_blocks(q, k_s, v_s, out_w, out_b):
    heads = []
    for h in range(NH):
        qh = q[:, h * HD:(h + 1) * HD]
        kh = k_s[:, h * HD:(h + 1) * HD]
        vh = v_s[:, h * HD:(h + 1) * HD]
        s = lax.dot_general(qh, kh, (((1,), (1,)), ((), ())),
                            preferred_element_type=jnp.float32) / _SQRT_HD
        m = jnp.max(s, axis=1, keepdims=True)
        e = jnp.exp(s - m)
        p = e / jnp.sum(e, axis=1, keepdims=True)
        oh = lax.dot_general(p, vh, (((1,), (0,)), ((), ())),
                             preferred_element_type=jnp.float32)
        heads.append(oh)
    o = jnp.concatenate(heads, axis=1)
    return lax.dot_general(o, out_w, (((1,), (1,)), ((), ())),
                           preferred_element_type=jnp.float32) + out_b


def _mha1_body(q_in_ref, kv_ref, in_w_ref, in_b_ref, out_w_ref, out_b_ref,
               o_ref, k_s, v_s):
    in_w = in_w_ref[...]
    in_b = in_b_ref[...]

    @pl.when(pl.program_id(0) == 0)
    def _():
        kv = kv_ref[...]
        k_s[...] = lax.dot_general(kv, in_w[HID:2 * HID, :],
                                   (((1,), (1,)), ((), ())),
                                   preferred_element_type=jnp.float32) \
            + in_b[:, HID:2 * HID]
        v_s[...] = lax.dot_general(kv, in_w[2 * HID:, :],
                                   (((1,), (1,)), ((), ())),
                                   preferred_element_type=jnp.float32) \
            + in_b[:, 2 * HID:]

    q = lax.dot_general(q_in_ref[...], in_w[:HID, :], (((1,), (1,)), ((), ())),
                        preferred_element_type=jnp.float32) + in_b[:, :HID]
    o_ref[...] = _mha_blocks(q, k_s[...], v_s[...], out_w_ref[...],
                             out_b_ref[...])


_MHA_BLK = 512


def _tc_mha(q_in, kv, in_w, in_b2, out_w, out_b2):
    lq = q_in.shape[0]
    lk = kv.shape[0]
    return pl.pallas_call(
        _mha1_body,
        grid=(lq // _MHA_BLK,),
        in_specs=[
            pl.BlockSpec((_MHA_BLK, HID), lambda i: (i, 0)),
            pl.BlockSpec((lk, HID), lambda i: (0, 0)),
            pl.BlockSpec((3 * HID, HID), lambda i: (0, 0)),
            pl.BlockSpec((1, 3 * HID), lambda i: (0, 0)),
            pl.BlockSpec((HID, HID), lambda i: (0, 0)),
            pl.BlockSpec((1, HID), lambda i: (0, 0)),
        ],
        out_specs=pl.BlockSpec((_MHA_BLK, HID), lambda i: (i, 0)),
        out_shape=jax.ShapeDtypeStruct((lq, HID), jnp.float32),
        scratch_shapes=[
            pltpu.VMEM((lk, HID), jnp.float32),
            pltpu.VMEM((lk, HID), jnp.float32),
        ],
    )(q_in, kv, in_w, in_b2, out_w, out_b2)


_RED_BLK = 512


def _reduce_mm_body(g_ref, x_ref, w_ref, o_ref):
    inter = jnp.sum(g_ref[...], axis=1)
    x2 = x_ref[...] + inter
    o_ref[...] = lax.dot_general(x2, w_ref[...], (((1,), (1,)), ((), ())),
                                 preferred_element_type=jnp.float32)


def _tc_reduce_mm(g3, x, w_hg):
    return pl.pallas_call(
        _reduce_mm_body,
        grid=(N // _RED_BLK,),
        in_specs=[
            pl.BlockSpec((_RED_BLK, K, HID), lambda i: (i, 0, 0)),
            pl.BlockSpec((_RED_BLK, HID), lambda i: (i, 0)),
            pl.BlockSpec((HID, HID), lambda i: (0, 0)),
        ],
        out_specs=pl.BlockSpec((_RED_BLK, HID), lambda i: (i, 0)),
        out_shape=jax.ShapeDtypeStruct((N, HID), jnp.float32),
    )(g3, x, w_hg)


def _combine_ef_body(e0_ref, e1_ref, bd0_ref, bd1_ref, o_ref):
    bd = bd0_ref[0][:, :1] + bd1_ref[0][:, :1]
    binv = jnp.where(bd > 0, 1.0 / bd, 0.0)
    o_ref[...] = (e0_ref[0] + e1_ref[0]) * binv


def _tc_combine_ef(ef_part, bd_part):
    blk = 512
    return pl.pallas_call(
        _combine_ef_body,
        grid=(NE // blk,),
        in_specs=[
            pl.BlockSpec((1, blk, HID), lambda i: (0, i, 0)),
            pl.BlockSpec((1, blk, HID), lambda i: (1, i, 0)),
            pl.BlockSpec((1, blk, HID), lambda i: (0, i, 0)),
            pl.BlockSpec((1, blk, HID), lambda i: (1, i, 0)),
        ],
        out_specs=pl.BlockSpec((blk, HID), lambda i: (i, 0)),
        out_shape=jax.ShapeDtypeStruct((NE, HID), jnp.float32),
    )(ef_part, ef_part, bd_part, bd_part)


def _final_body(op_ref, dp_ref, bhg_ref, gam_ref, bet_ref, m_ref,
                in_w_ref, in_b_ref, out_w_ref, out_b_ref, o_ref, k_s, v_s):
    in_w = in_w_ref[...]
    in_b = in_b_ref[...]

    @pl.when(pl.program_id(0) == 0)
    def _():
        d = dp_ref[0, :, :1] + dp_ref[1, :, :1]
        dinv = jnp.where(d > 0, 1.0 / d, 0.0)
        out = (op_ref[0] + op_ref[1]) * dinv + bhg_ref[...]
        mu = jnp.mean(out, axis=0, keepdims=True)
        var = jnp.mean((out - mu) ** 2, axis=0, keepdims=True)
        out = gam_ref[...] * (out - mu) / jnp.sqrt(var + 1e-5) + bet_ref[...]
        out = jnp.where(out > 0, out, jnp.exp(jnp.minimum(out, 0.0)) - 1.0)
        k_s[...] = lax.dot_general(out, in_w[HID:2 * HID, :],
                                   (((1,), (1,)), ((), ())),
                                   preferred_element_type=jnp.float32) \
            + in_b[:, HID:2 * HID]
        v_s[...] = lax.dot_general(out, in_w[2 * HID:, :],
                                   (((1,), (1,)), ((), ())),
                                   preferred_element_type=jnp.float32) \
            + in_b[:, 2 * HID:]

    q = lax.dot_general(m_ref[...], in_w[:HID, :], (((1,), (1,)), ((), ())),
                        preferred_element_type=jnp.float32) + in_b[:, :HID]
    o_ref[...] = _mha_blocks(q, k_s[...], v_s[...], out_w_ref[...],
                             out_b_ref[...])


def _tc_final(out_part, d_part, b_hg2, gamma2, beta2, m_embs,
              in_w, in_b2, out_w, out_b2):
    return pl.pallas_call(
        _final_body,
        grid=(Q // _MHA_BLK,),
        in_specs=[
            pl.BlockSpec((2, N, HID), lambda i: (0, 0, 0)),
            pl.BlockSpec((2, N, HID), lambda i: (0, 0, 0)),
            pl.BlockSpec((1, HID), lambda i: (0, 0)),
            pl.BlockSpec((1, HID), lambda i: (0, 0)),
            pl.BlockSpec((1, HID), lambda i: (0, 0)),
            pl.BlockSpec((_MHA_BLK, HID), lambda i: (i, 0)),
            pl.BlockSpec((3 * HID, HID), lambda i: (0, 0)),
            pl.BlockSpec((1, 3 * HID), lambda i: (0, 0)),
            pl.BlockSpec((HID, HID), lambda i: (0, 0)),
            pl.BlockSpec((1, HID), lambda i: (0, 0)),
        ],
        out_specs=pl.BlockSpec((_MHA_BLK, HID), lambda i: (i, 0)),
        out_shape=jax.ShapeDtypeStruct((Q, HID), jnp.float32),
        scratch_shapes=[
            pltpu.VMEM((N, HID), jnp.float32),
            pltpu.VMEM((N, HID), jnp.float32),
        ],
    )(out_part, d_part, b_hg2, gamma2, beta2, m_embs, in_w, in_b2, out_w,
      out_b2)


def _sc_mesh():
    return plsc.VectorSubcoreMesh(core_axis_name="c", subcore_axis_name="s",
                                  num_cores=NC, num_subcores=NS)
_CHUNK = 512
_CCHUNK = 128
_EDGE_PER_TILE = EDGES // (NC * NS)
_EDGE_CHUNKS = _EDGE_PER_TILE // _CHUNK
_STRIPE = N // NS


def _sc_counts(nodes, edges, zeros128, ones128):
    @functools.partial(
        pl.kernel, mesh=_sc_mesh(),
        out_type=[
            jax.ShapeDtypeStruct((NC, N, HID), jnp.float32),
            jax.ShapeDtypeStruct((NC, NE, HID), jnp.float32),
        ],
        scratch_types=[
            pltpu.VMEM((_CCHUNK,), jnp.int32),
            pltpu.VMEM((_CCHUNK, HID), jnp.float32),
            pltpu.VMEM_SHARED((N, HID), jnp.float32),
            pltpu.VMEM_SHARED((NE, HID), jnp.float32),
        ],
    )
    def k(nodes_hbm, edges_hbm, z_hbm, ones_hbm, d_out, bd_out,
          idx_v, ones_v, d_sh, bd_sh):
        c = lax.axis_index("c")
        s = lax.axis_index("s")
        stripe = s * _STRIPE
        pltpu.sync_copy(z_hbm.at[pl.ds(stripe, _STRIPE)],
                        d_sh.at[pl.ds(stripe, _STRIPE)])
        pltpu.sync_copy(z_hbm.at[pl.ds(stripe, _STRIPE)],
                        bd_sh.at[pl.ds(stripe, _STRIPE)])
        pltpu.sync_copy(ones_hbm, ones_v)
        plsc.subcore_barrier()
        base = c * (EDGES // NC) + s * _EDGE_PER_TILE
        for j in range(_EDGE_PER_TILE // _CCHUNK):
            pltpu.sync_copy(nodes_hbm.at[pl.ds(base + j * _CCHUNK, _CCHUNK)],
                            idx_v)
            pltpu.sync_copy(ones_v, d_sh.at[idx_v], add=True)
            pltpu.sync_copy(edges_hbm.at[pl.ds(base + j * _CCHUNK, _CCHUNK)],
                            idx_v)
            pltpu.sync_copy(ones_v, bd_sh.at[idx_v], add=True)
        plsc.subcore_barrier()
        pltpu.sync_copy(d_sh.at[pl.ds(stripe, _STRIPE)],
                        d_out.at[c].at[pl.ds(stripe, _STRIPE)])
        pltpu.sync_copy(bd_sh.at[pl.ds(stripe, _STRIPE)],
                        bd_out.at[c].at[pl.ds(stripe, _STRIPE)])

    return k(nodes, edges, zeros128, ones128)


def _sc_gather(table, idx):
    b = idx.shape[0]
    per_w = b // (NC * NS)
    chunk = 256
    nch = per_w // chunk

    @functools.partial(
        pl.kernel, mesh=_sc_mesh(),
        out_type=jax.ShapeDtypeStruct((b, HID), jnp.float32),
        scratch_types=[
            pltpu.VMEM((chunk,), jnp.int32),
            pltpu.VMEM((chunk, HID), jnp.float32),
            pltpu.SemaphoreType.DMA,
        ],
    )
    def k(tab_hbm, idx_hbm, o_hbm, idx_v, rows_v, sem):
        c = lax.axis_index("c")
        s = lax.axis_index("s")
        wid = s * NC + c
        base = wid * per_w
        for j in range(nch):
            off = base + j * chunk
            pltpu.sync_copy(idx_hbm.at[pl.ds(off, chunk)], idx_v)
            pltpu.async_copy(tab_hbm.at[idx_v], rows_v, sem).wait()
            pltpu.sync_copy(rows_v, o_hbm.at[pl.ds(off, chunk)])

    return k(table, idx)


def _sc_segsum(values, gather_idx, scatter_idx, zeros128):
    @functools.partial(
        pl.kernel, mesh=_sc_mesh(),
        out_type=jax.ShapeDtypeStruct((NC, N, HID), jnp.float32),
        scratch_types=[
            pltpu.VMEM((_CHUNK,), jnp.int32),
            pltpu.VMEM((_CHUNK,), jnp.int32),
            pltpu.VMEM((_CHUNK, HID), jnp.float32),
            pltpu.VMEM_SHARED((N, HID), jnp.float32),
            pltpu.SemaphoreType.DMA,
        ],
    )
    def k(val_hbm, gi_hbm, si_hbm, z_hbm, o_hbm, gi_v, si_v, rows_v, acc_sh,
          sem):
        c = lax.axis_index("c")
        s = lax.axis_index("s")
        stripe = s * _STRIPE
        pltpu.sync_copy(z_hbm.at[pl.ds(stripe, _STRIPE)],
                        acc_sh.at[pl.ds(stripe, _STRIPE)])
        plsc.subcore_barrier()
        base = c * (EDGES // NC) + s * _EDGE_PER_TILE
        for j in range(_EDGE_CHUNKS):
            off = base + j * _CHUNK
            pltpu.sync_copy(gi_hbm.at[pl.ds(off, _CHUNK)], gi_v)
            pltpu.sync_copy(si_hbm.at[pl.ds(off, _CHUNK)], si_v)
            pltpu.async_copy(val_hbm.at[gi_v], rows_v, sem).wait()
            pltpu.sync_copy(rows_v, acc_sh.at[si_v], add=True)
        plsc.subcore_barrier()
        pltpu.sync_copy(acc_sh.at[pl.ds(stripe, _STRIPE)],
                        o_hbm.at[c].at[pl.ds(stripe, _STRIPE)])

    return k(values, gather_idx, scatter_idx, zeros128)


def kernel(embs1, embs2, m_embs, edge_index, W_hg, b_hg, bn_gamma, bn_beta,
           attn_in_w, attn_in_b, attn_out_w, attn_out_b,
           mha_in_w, mha_in_b, mha_out_w, mha_out_b):
    x = jnp.concatenate([embs1, embs2], axis=0)
    nodes = edge_index[0]
    edges = edge_index[1]
    he = jax.random.normal(jax.random.key(1), (N, HID), dtype=jnp.float32)

    zeros128 = jnp.zeros((N, HID), jnp.float32)
    ones128 = jnp.ones((_CCHUNK, HID), jnp.float32)

    d_part, bd_part = _sc_counts(nodes, edges, zeros128, ones128)

    nearest = _tc_knn(x)
    he_out = _tc_mha(he, x, attn_in_w, attn_in_b.reshape(1, -1),
                     attn_out_w, attn_out_b.reshape(1, -1))

    idx_flat = nearest[:, :K].reshape(-1)
    g = _sc_gather(he_out, idx_flat)
    xw = _tc_reduce_mm(g.reshape(N, K, HID), x, W_hg)

    ef_part = _sc_segsum(xw, nodes, edges, zeros128)
    ef = _tc_combine_ef(ef_part, bd_part)
    out_part = _sc_segsum(ef, edges, nodes, zeros128)

    return _tc_final(out_part, d_part, b_hg.reshape(1, -1),
                     bn_gamma.reshape(1, -1), bn_beta.reshape(1, -1),
                     m_embs, mha_in_w, mha_in_b.reshape(1, -1),
                     mha_out_w, mha_out_b.reshape(1, -1))


# speedup vs baseline: 6.8407x; 1.2079x over previous
import functools

import jax
import jax.numpy as jnp
import numpy as np
from jax import lax
from jax.experimental import pallas as pl
from jax.experimental.pallas import tpu as pltpu
from jax.experimental.pallas import tpu_sc as plsc

HID = 128
NH = 4
HD = HID // NH
K = 10
N = 4096
Q = 2048
EDGES = 65536
NE = 4096

NC = 2
NS = 16

_SQRT_HD = np.sqrt(np.float32(HD)).astype(np.float32)


_KNN_BLK = 256


def _knn_body(xb_ref, xf_ref, out_ref):
    xb = xb_ref[...]
    xf = xf_ref[...]
    sq_i = jnp.sum(xb * xb, axis=1, keepdims=True)
    sq_j = jnp.sum(xf * xf, axis=1)
    s = lax.dot_general(xb, xf, (((1,), (1,)), ((), ())),
                        preferred_element_type=jnp.float32)
    d2 = sq_i + sq_j[None, :] - 2.0 * s
    cur = jnp.maximum(d2, 0.0)
    col = lax.broadcasted_iota(jnp.int32, cur.shape, 1)
    cols = []
    for t in range(K):
        idx = jnp.argmin(cur, axis=1).astype(jnp.int32)
        cols.append(idx)
        if t < K - 1:
            cur = jnp.where(col == idx[:, None], jnp.inf, cur)
    for _ in range(16 - K):
        cols.append(cols[-1])
    out_ref[...] = jnp.stack(cols, axis=1)


def _tc_knn(x):
    return pl.pallas_call(
        _knn_body,
        grid=(N // _KNN_BLK,),
        in_specs=[
            pl.BlockSpec((_KNN_BLK, HID), lambda i: (i, 0)),
            pl.BlockSpec((N, HID), lambda i: (0, 0)),
        ],
        out_specs=pl.BlockSpec((_KNN_BLK, 16), lambda i: (i, 0)),
        out_shape=jax.ShapeDtypeStruct((N, 16), jnp.int32),
    )(x, x)


def _mha_blocks(q, k_s, v_s, out_w, out_b):
    heads = []
    for h in range(NH):
        qh = q[:, h * HD:(h + 1) * HD] * (1.0 / _SQRT_HD)
        kh = k_s[:, h * HD:(h + 1) * HD]
        vh = v_s[:, h * HD:(h + 1) * HD]
        s = lax.dot_general(qh, kh, (((1,), (1,)), ((), ())),
                            preferred_element_type=jnp.float32)
        m = jnp.max(s, axis=1, keepdims=True)
        e = jnp.exp(s - m)
        oh = lax.dot_general(e, vh, (((1,), (0,)), ((), ())),
                             preferred_element_type=jnp.float32)
        oh = oh / jnp.sum(e, axis=1, keepdims=True)
        heads.append(oh)
    o = jnp.concatenate(heads, axis=1)
    return lax.dot_general(o, out_w, (((1,), (1,)), ((), ())),
                           preferred_element_type=jnp.float32) + out_b


def _mha1_body(q_in_ref, kv_ref, in_w_ref, in_b_ref, out_w_ref, out_b_ref,
               o_ref, k_s, v_s):
    in_w = in_w_ref[...]
    in_b = in_b_ref[...]

    @pl.when(pl.program_id(0) == 0)
    def _():
        kv = kv_ref[...]
        k_s[...] = lax.dot_general(kv, in_w[HID:2 * HID, :],
                                   (((1,), (1,)), ((), ())),
                                   preferred_element_type=jnp.float32) \
            + in_b[:, HID:2 * HID]
        v_s[...] = lax.dot_general(kv, in_w[2 * HID:, :],
                                   (((1,), (1,)), ((), ())),
                                   preferred_element_type=jnp.float32) \
            + in_b[:, 2 * HID:]

    q = lax.dot_general(q_in_ref[...], in_w[:HID, :], (((1,), (1,)), ((), ())),
                        preferred_element_type=jnp.float32) + in_b[:, :HID]
    o_ref[...] = _mha_blocks(q, k_s[...], v_s[...], out_w_ref[...],
                             out_b_ref[...])


_MHA_BLK = 512


def _tc_mha(q_in, kv, in_w, in_b2, out_w, out_b2):
    lq = q_in.shape[0]
    lk = kv.shape[0]
    return pl.pallas_call(
        _mha1_body,
        grid=(lq // _MHA_BLK,),
        in_specs=[
            pl.BlockSpec((_MHA_BLK, HID), lambda i: (i, 0)),
            pl.BlockSpec((lk, HID), lambda i: (0, 0)),
            pl.BlockSpec((3 * HID, HID), lambda i: (0, 0)),
            pl.BlockSpec((1, 3 * HID), lambda i: (0, 0)),
            pl.BlockSpec((HID, HID), lambda i: (0, 0)),
            pl.BlockSpec((1, HID), lambda i: (0, 0)),
        ],
        out_specs=pl.BlockSpec((_MHA_BLK, HID), lambda i: (i, 0)),
        out_shape=jax.ShapeDtypeStruct((lq, HID), jnp.float32),
        scratch_shapes=[
            pltpu.VMEM((lk, HID), jnp.float32),
            pltpu.VMEM((lk, HID), jnp.float32),
        ],
    )(q_in, kv, in_w, in_b2, out_w, out_b2)


_RED_BLK = 512


def _reduce_mm_body(g_ref, x_ref, w_ref, o_ref):
    inter = jnp.sum(g_ref[...], axis=1)
    x2 = x_ref[...] + inter
    o_ref[...] = lax.dot_general(x2, w_ref[...], (((1,), (1,)), ((), ())),
                                 preferred_element_type=jnp.float32)


def _tc_reduce_mm(g3, x, w_hg):
    return pl.pallas_call(
        _reduce_mm_body,
        grid=(N // _RED_BLK,),
        in_specs=[
            pl.BlockSpec((_RED_BLK, K, HID), lambda i: (i, 0, 0)),
            pl.BlockSpec((_RED_BLK, HID), lambda i: (i, 0)),
            pl.BlockSpec((HID, HID), lambda i: (0, 0)),
        ],
        out_specs=pl.BlockSpec((_RED_BLK, HID), lambda i: (i, 0)),
        out_shape=jax.ShapeDtypeStruct((N, HID), jnp.float32),
    )(g3, x, w_hg)


def _combine_ef_body(e0_ref, e1_ref, bd0_ref, bd1_ref, o_ref):
    bd = bd0_ref[0][:, :1] + bd1_ref[0][:, :1]
    binv = jnp.where(bd > 0, 1.0 / bd, 0.0)
    o_ref[...] = (e0_ref[0] + e1_ref[0]) * binv


def _tc_combine_ef(ef_part, bd_part):
    blk = 512
    return pl.pallas_call(
        _combine_ef_body,
        grid=(NE // blk,),
        in_specs=[
            pl.BlockSpec((1, blk, HID), lambda i: (0, i, 0)),
            pl.BlockSpec((1, blk, HID), lambda i: (1, i, 0)),
            pl.BlockSpec((1, blk, HID), lambda i: (0, i, 0)),
            pl.BlockSpec((1, blk, HID), lambda i: (1, i, 0)),
        ],
        out_specs=pl.BlockSpec((blk, HID), lambda i: (i, 0)),
        out_shape=jax.ShapeDtypeStruct((NE, HID), jnp.float32),
    )(ef_part, ef_part, bd_part, bd_part)


def _final_body(op_ref, dp_ref, bhg_ref, gam_ref, bet_ref, m_ref,
                in_w_ref, in_b_ref, out_w_ref, out_b_ref, o_ref, k_s, v_s):
    in_w = in_w_ref[...]
    in_b = in_b_ref[...]

    @pl.when(pl.program_id(0) == 0)
    def _():
        d = dp_ref[0, :, :1] + dp_ref[1, :, :1]
        dinv = jnp.where(d > 0, 1.0 / d, 0.0)
        out = (op_ref[0] + op_ref[1]) * dinv + bhg_ref[...]
        mu = jnp.mean(out, axis=0, keepdims=True)
        var = jnp.mean((out - mu) ** 2, axis=0, keepdims=True)
        out = gam_ref[...] * (out - mu) / jnp.sqrt(var + 1e-5) + bet_ref[...]
        out = jnp.where(out > 0, out, jnp.exp(jnp.minimum(out, 0.0)) - 1.0)
        k_s[...] = lax.dot_general(out, in_w[HID:2 * HID, :],
                                   (((1,), (1,)), ((), ())),
                                   preferred_element_type=jnp.float32) \
            + in_b[:, HID:2 * HID]
        v_s[...] = lax.dot_general(out, in_w[2 * HID:, :],
                                   (((1,), (1,)), ((), ())),
                                   preferred_element_type=jnp.float32) \
            + in_b[:, 2 * HID:]

    q = lax.dot_general(m_ref[...], in_w[:HID, :], (((1,), (1,)), ((), ())),
                        preferred_element_type=jnp.float32) + in_b[:, :HID]
    o_ref[...] = _mha_blocks(q, k_s[...], v_s[...], out_w_ref[...],
                             out_b_ref[...])


def _tc_final(out_part, d_part, b_hg2, gamma2, beta2, m_embs,
              in_w, in_b2, out_w, out_b2):
    return pl.pallas_call(
        _final_body,
        grid=(Q // _MHA_BLK,),
        in_specs=[
            pl.BlockSpec((2, N, HID), lambda i: (0, 0, 0)),
            pl.BlockSpec((2, N, HID), lambda i: (0, 0, 0)),
            pl.BlockSpec((1, HID), lambda i: (0, 0)),
            pl.BlockSpec((1, HID), lambda i: (0, 0)),
            pl.BlockSpec((1, HID), lambda i: (0, 0)),
            pl.BlockSpec((_MHA_BLK, HID), lambda i: (i, 0)),
            pl.BlockSpec((3 * HID, HID), lambda i: (0, 0)),
            pl.BlockSpec((1, 3 * HID), lambda i: (0, 0)),
            pl.BlockSpec((HID, HID), lambda i: (0, 0)),
            pl.BlockSpec((1, HID), lambda i: (0, 0)),
        ],
        out_specs=pl.BlockSpec((_MHA_BLK, HID), lambda i: (i, 0)),
        out_shape=jax.ShapeDtypeStruct((Q, HID), jnp.float32),
        scratch_shapes=[
            pltpu.VMEM((N, HID), jnp.float32),
            pltpu.VMEM((N, HID), jnp.float32),
        ],
    )(out_part, d_part, b_hg2, gamma2, beta2, m_embs, in_w, in_b2, out_w,
      out_b2)


def _sc_mesh():
    return plsc.VectorSubcoreMesh(core_axis_name="c", subcore_axis_name="s",
                                  num_cores=NC, num_subcores=NS)
_CHUNK = 512
_CCHUNK = 128
_EDGE_PER_TILE = EDGES // (NC * NS)
_EDGE_CHUNKS = _EDGE_PER_TILE // _CHUNK
_STRIPE = N // NS


def _sc_counts(nodes, edges, zeros128, ones128):
    @functools.partial(
        pl.kernel, mesh=_sc_mesh(),
        out_type=[
            jax.ShapeDtypeStruct((NC, N, HID), jnp.float32),
            jax.ShapeDtypeStruct((NC, NE, HID), jnp.float32),
        ],
        scratch_types=[
            pltpu.VMEM((_CCHUNK,), jnp.int32),
            pltpu.VMEM((_CCHUNK, HID), jnp.float32),
            pltpu.VMEM_SHARED((N, HID), jnp.float32),
            pltpu.VMEM_SHARED((NE, HID), jnp.float32),
        ],
    )
    def k(nodes_hbm, edges_hbm, z_hbm, ones_hbm, d_out, bd_out,
          idx_v, ones_v, d_sh, bd_sh):
        c = lax.axis_index("c")
        s = lax.axis_index("s")
        stripe = s * _STRIPE
        pltpu.sync_copy(z_hbm.at[pl.ds(stripe, _STRIPE)],
                        d_sh.at[pl.ds(stripe, _STRIPE)])
        pltpu.sync_copy(z_hbm.at[pl.ds(stripe, _STRIPE)],
                        bd_sh.at[pl.ds(stripe, _STRIPE)])
        pltpu.sync_copy(ones_hbm, ones_v)
        plsc.subcore_barrier()
        base = c * (EDGES // NC) + s * _EDGE_PER_TILE
        for j in range(_EDGE_PER_TILE // _CCHUNK):
            pltpu.sync_copy(nodes_hbm.at[pl.ds(base + j * _CCHUNK, _CCHUNK)],
                            idx_v)
            pltpu.sync_copy(ones_v, d_sh.at[idx_v], add=True)
            pltpu.sync_copy(edges_hbm.at[pl.ds(base + j * _CCHUNK, _CCHUNK)],
                            idx_v)
            pltpu.sync_copy(ones_v, bd_sh.at[idx_v], add=True)
        plsc.subcore_barrier()
        pltpu.sync_copy(d_sh.at[pl.ds(stripe, _STRIPE)],
                        d_out.at[c].at[pl.ds(stripe, _STRIPE)])
        pltpu.sync_copy(bd_sh.at[pl.ds(stripe, _STRIPE)],
                        bd_out.at[c].at[pl.ds(stripe, _STRIPE)])

    return k(nodes, edges, zeros128, ones128)


def _sc_gather(table, idx):
    b = idx.shape[0]
    per_w = b // (NC * NS)
    chunk = 256
    nch = per_w // chunk

    @functools.partial(
        pl.kernel, mesh=_sc_mesh(),
        out_type=jax.ShapeDtypeStruct((b, HID), jnp.float32),
        scratch_types=[
            pltpu.VMEM((chunk,), jnp.int32),
            pltpu.VMEM((chunk, HID), jnp.float32),
            pltpu.SemaphoreType.DMA,
        ],
    )
    def k(tab_hbm, idx_hbm, o_hbm, idx_v, rows_v, sem):
        c = lax.axis_index("c")
        s = lax.axis_index("s")
        wid = s * NC + c
        base = wid * per_w
        for j in range(nch):
            off = base + j * chunk
            pltpu.sync_copy(idx_hbm.at[pl.ds(off, chunk)], idx_v)
            pltpu.async_copy(tab_hbm.at[idx_v], rows_v, sem).wait()
            pltpu.sync_copy(rows_v, o_hbm.at[pl.ds(off, chunk)])

    return k(table, idx)


def _sc_segsum(values, gather_idx, scatter_idx, zeros128):
    @functools.partial(
        pl.kernel, mesh=_sc_mesh(),
        out_type=jax.ShapeDtypeStruct((NC, N, HID), jnp.float32),
        scratch_types=[
            pltpu.VMEM((_CHUNK,), jnp.int32),
            pltpu.VMEM((_CHUNK,), jnp.int32),
            pltpu.VMEM((_CHUNK, HID), jnp.float32),
            pltpu.VMEM_SHARED((N, HID), jnp.float32),
            pltpu.SemaphoreType.DMA,
        ],
    )
    def k(val_hbm, gi_hbm, si_hbm, z_hbm, o_hbm, gi_v, si_v, rows_v, acc_sh,
          sem):
        c = lax.axis_index("c")
        s = lax.axis_index("s")
        stripe = s * _STRIPE
        pltpu.sync_copy(z_hbm.at[pl.ds(stripe, _STRIPE)],
                        acc_sh.at[pl.ds(stripe, _STRIPE)])
        plsc.subcore_barrier()
        base = c * (EDGES // NC) + s * _EDGE_PER_TILE
        for j in range(_EDGE_CHUNKS):
            off = base + j * _CHUNK
            pltpu.sync_copy(gi_hbm.at[pl.ds(off, _CHUNK)], gi_v)
            pltpu.sync_copy(si_hbm.at[pl.ds(off, _CHUNK)], si_v)
            pltpu.async_copy(val_hbm.at[gi_v], rows_v, sem).wait()
            pltpu.sync_copy(rows_v, acc_sh.at[si_v], add=True)
        plsc.subcore_barrier()
        pltpu.sync_copy(acc_sh.at[pl.ds(stripe, _STRIPE)],
                        o_hbm.at[c].at[pl.ds(stripe, _STRIPE)])

    return k(values, gather_idx, scatter_idx, zeros128)


def kernel(embs1, embs2, m_embs, edge_index, W_hg, b_hg, bn_gamma, bn_beta,
           attn_in_w, attn_in_b, attn_out_w, attn_out_b,
           mha_in_w, mha_in_b, mha_out_w, mha_out_b):
    x = jnp.concatenate([embs1, embs2], axis=0)
    nodes = edge_index[0]
    edges = edge_index[1]
    he = jax.random.normal(jax.random.key(1), (N, HID), dtype=jnp.float32)

    zeros128 = jnp.zeros((N, HID), jnp.float32)
    ones128 = jnp.ones((_CCHUNK, HID), jnp.float32)

    d_part, bd_part = _sc_counts(nodes, edges, zeros128, ones128)

    nearest = _tc_knn(x)
    he_out = _tc_mha(he, x, attn_in_w, attn_in_b.reshape(1, -1),
                     attn_out_w, attn_out_b.reshape(1, -1))

    idx_flat = nearest[:, :K].reshape(-1)
    g = _sc_gather(he_out, idx_flat)
    xw = _tc_reduce_mm(g.reshape(N, K, HID), x, W_hg)

    ef_part = _sc_segsum(xw, nodes, edges, zeros128)
    ef = _tc_combine_ef(ef_part, bd_part)
    out_part = _sc_segsum(ef, edges, nodes, zeros128)

    return _tc_final(out_part, d_part, b_hg.reshape(1, -1),
                     bn_gamma.reshape(1, -1), bn_beta.reshape(1, -1),
                     m_embs, mha_in_w, mha_in_b.reshape(1, -1),
                     mha_out_w, mha_out_b.reshape(1, -1))


# speedup vs baseline: 7.9310x; 1.1594x over previous
import functools

import jax
import jax.numpy as jnp
import numpy as np
from jax import lax
from jax.experimental import pallas as pl
from jax.experimental.pallas import tpu as pltpu
from jax.experimental.pallas import tpu_sc as plsc

HID = 128
NH = 4
HD = HID // NH
K = 10
N = 4096
Q = 2048
EDGES = 65536
NE = 4096

NC = 2
NS = 16

_SQRT_HD = np.sqrt(np.float32(HD)).astype(np.float32)


_KNN_BLK = 256


def _knn_body(xb_ref, xf_ref, he_ref, w_ref, out_ref):
    xb = xb_ref[...]
    xf = xf_ref[...]
    sq_i = jnp.sum(xb * xb, axis=1)
    sq_j = jnp.sum(xf * xf, axis=1, keepdims=True)
    s = lax.dot_general(xf, xb, (((1,), (1,)), ((), ())),
                        preferred_element_type=jnp.float32)
    d2 = sq_j + sq_i[None, :] - 2.0 * s
    cur = jnp.maximum(d2, 0.0)
    inf = jnp.float32(jnp.inf)
    for _ in range(K):
        m = jnp.min(cur, axis=0)
        cur = jnp.where(cur == m[None, :], inf, cur)
    mask = jnp.where(cur == inf, 1.0, 0.0)
    inter = lax.dot_general(mask, he_ref[...], (((0,), (0,)), ((), ())),
                            preferred_element_type=jnp.float32)
    x2 = xb + inter
    out_ref[...] = lax.dot_general(x2, w_ref[...], (((1,), (1,)), ((), ())),
                                   preferred_element_type=jnp.float32)


def _tc_knn_mm(x, he_out, w_hg):
    return pl.pallas_call(
        _knn_body,
        grid=(N // _KNN_BLK,),
        in_specs=[
            pl.BlockSpec((_KNN_BLK, HID), lambda i: (i, 0)),
            pl.BlockSpec((N, HID), lambda i: (0, 0)),
            pl.BlockSpec((N, HID), lambda i: (0, 0)),
            pl.BlockSpec((HID, HID), lambda i: (0, 0)),
        ],
        out_specs=pl.BlockSpec((_KNN_BLK, HID), lambda i: (i, 0)),
        out_shape=jax.ShapeDtypeStruct((N, HID), jnp.float32),
    )(x, x, he_out, w_hg)


def _mha_blocks(q, k_s, v_s, out_w, out_b):
    heads = []
    for h in range(NH):
        qh = q[:, h * HD:(h + 1) * HD] * (1.0 / _SQRT_HD)
        kh = k_s[:, h * HD:(h + 1) * HD]
        vh = v_s[:, h * HD:(h + 1) * HD]
        s = lax.dot_general(qh, kh, (((1,), (1,)), ((), ())),
                            preferred_element_type=jnp.float32)
        m = jnp.max(s, axis=1, keepdims=True)
        e = jnp.exp(s - m)
        oh = lax.dot_general(e, vh, (((1,), (0,)), ((), ())),
                             preferred_element_type=jnp.float32)
        oh = oh / jnp.sum(e, axis=1, keepdims=True)
        heads.append(oh)
    o = jnp.concatenate(heads, axis=1)
    return lax.dot_general(o, out_w, (((1,), (1,)), ((), ())),
                           preferred_element_type=jnp.float32) + out_b


def _mha1_body(q_in_ref, kv_ref, in_w_ref, in_b_ref, out_w_ref, out_b_ref,
               o_ref, k_s, v_s):
    in_w = in_w_ref[...]
    in_b = in_b_ref[...]

    @pl.when(pl.program_id(0) == 0)
    def _():
        kv = kv_ref[...]
        k_s[...] = lax.dot_general(kv, in_w[HID:2 * HID, :],
                                   (((1,), (1,)), ((), ())),
                                   preferred_element_type=jnp.float32) \
            + in_b[:, HID:2 * HID]
        v_s[...] = lax.dot_general(kv, in_w[2 * HID:, :],
                                   (((1,), (1,)), ((), ())),
                                   preferred_element_type=jnp.float32) \
            + in_b[:, 2 * HID:]

    q = lax.dot_general(q_in_ref[...], in_w[:HID, :], (((1,), (1,)), ((), ())),
                        preferred_element_type=jnp.float32) + in_b[:, :HID]
    o_ref[...] = _mha_blocks(q, k_s[...], v_s[...], out_w_ref[...],
                             out_b_ref[...])


_MHA_BLK = 512


def _tc_mha(q_in, kv, in_w, in_b2, out_w, out_b2):
    lq = q_in.shape[0]
    lk = kv.shape[0]
    return pl.pallas_call(
        _mha1_body,
        grid=(lq // _MHA_BLK,),
        in_specs=[
            pl.BlockSpec((_MHA_BLK, HID), lambda i: (i, 0)),
            pl.BlockSpec((lk, HID), lambda i: (0, 0)),
            pl.BlockSpec((3 * HID, HID), lambda i: (0, 0)),
            pl.BlockSpec((1, 3 * HID), lambda i: (0, 0)),
            pl.BlockSpec((HID, HID), lambda i: (0, 0)),
            pl.BlockSpec((1, HID), lambda i: (0, 0)),
        ],
        out_specs=pl.BlockSpec((_MHA_BLK, HID), lambda i: (i, 0)),
        out_shape=jax.ShapeDtypeStruct((lq, HID), jnp.float32),
        scratch_shapes=[
            pltpu.VMEM((lk, HID), jnp.float32),
            pltpu.VMEM((lk, HID), jnp.float32),
        ],
    )(q_in, kv, in_w, in_b2, out_w, out_b2)


def _combine_ef_body(e0_ref, e1_ref, bd0_ref, bd1_ref, o_ref):
    bd = bd0_ref[0][:, :1] + bd1_ref[0][:, :1]
    binv = jnp.where(bd > 0, 1.0 / bd, 0.0)
    o_ref[...] = (e0_ref[0] + e1_ref[0]) * binv


def _tc_combine_ef(ef_part, bd_part):
    blk = 512
    return pl.pallas_call(
        _combine_ef_body,
        grid=(NE // blk,),
        in_specs=[
            pl.BlockSpec((1, blk, HID), lambda i: (0, i, 0)),
            pl.BlockSpec((1, blk, HID), lambda i: (1, i, 0)),
            pl.BlockSpec((1, blk, HID), lambda i: (0, i, 0)),
            pl.BlockSpec((1, blk, HID), lambda i: (1, i, 0)),
        ],
        out_specs=pl.BlockSpec((blk, HID), lambda i: (i, 0)),
        out_shape=jax.ShapeDtypeStruct((NE, HID), jnp.float32),
    )(ef_part, ef_part, bd_part, bd_part)


def _final_body(op_ref, dp_ref, bhg_ref, gam_ref, bet_ref, m_ref,
                in_w_ref, in_b_ref, out_w_ref, out_b_ref, o_ref, k_s, v_s):
    in_w = in_w_ref[...]
    in_b = in_b_ref[...]

    @pl.when(pl.program_id(0) == 0)
    def _():
        d = dp_ref[0, :, :1] + dp_ref[1, :, :1]
        dinv = jnp.where(d > 0, 1.0 / d, 0.0)
        out = (op_ref[0] + op_ref[1]) * dinv + bhg_ref[...]
        mu = jnp.mean(out, axis=0, keepdims=True)
        var = jnp.mean((out - mu) ** 2, axis=0, keepdims=True)
        out = gam_ref[...] * (out - mu) / jnp.sqrt(var + 1e-5) + bet_ref[...]
        out = jnp.where(out > 0, out, jnp.exp(jnp.minimum(out, 0.0)) - 1.0)
        k_s[...] = lax.dot_general(out, in_w[HID:2 * HID, :],
                                   (((1,), (1,)), ((), ())),
                                   preferred_element_type=jnp.float32) \
            + in_b[:, HID:2 * HID]
        v_s[...] = lax.dot_general(out, in_w[2 * HID:, :],
                                   (((1,), (1,)), ((), ())),
                                   preferred_element_type=jnp.float32) \
            + in_b[:, 2 * HID:]

    q = lax.dot_general(m_ref[...], in_w[:HID, :], (((1,), (1,)), ((), ())),
                        preferred_element_type=jnp.float32) + in_b[:, :HID]
    o_ref[...] = _mha_blocks(q, k_s[...], v_s[...], out_w_ref[...],
                             out_b_ref[...])


def _tc_final(out_part, d_part, b_hg2, gamma2, beta2, m_embs,
              in_w, in_b2, out_w, out_b2):
    return pl.pallas_call(
        _final_body,
        grid=(Q // _MHA_BLK,),
        in_specs=[
            pl.BlockSpec((2, N, HID), lambda i: (0, 0, 0)),
            pl.BlockSpec((2, N, HID), lambda i: (0, 0, 0)),
            pl.BlockSpec((1, HID), lambda i: (0, 0)),
            pl.BlockSpec((1, HID), lambda i: (0, 0)),
            pl.BlockSpec((1, HID), lambda i: (0, 0)),
            pl.BlockSpec((_MHA_BLK, HID), lambda i: (i, 0)),
            pl.BlockSpec((3 * HID, HID), lambda i: (0, 0)),
            pl.BlockSpec((1, 3 * HID), lambda i: (0, 0)),
            pl.BlockSpec((HID, HID), lambda i: (0, 0)),
            pl.BlockSpec((1, HID), lambda i: (0, 0)),
        ],
        out_specs=pl.BlockSpec((_MHA_BLK, HID), lambda i: (i, 0)),
        out_shape=jax.ShapeDtypeStruct((Q, HID), jnp.float32),
        scratch_shapes=[
            pltpu.VMEM((N, HID), jnp.float32),
            pltpu.VMEM((N, HID), jnp.float32),
        ],
    )(out_part, d_part, b_hg2, gamma2, beta2, m_embs, in_w, in_b2, out_w,
      out_b2)


def _sc_mesh():
    return plsc.VectorSubcoreMesh(core_axis_name="c", subcore_axis_name="s",
                                  num_cores=NC, num_subcores=NS)
_CHUNK = 512
_CCHUNK = 128
_EDGE_PER_TILE = EDGES // (NC * NS)
_EDGE_CHUNKS = _EDGE_PER_TILE // _CHUNK
_STRIPE = N // NS


def _sc_counts(nodes, edges, zeros128, ones128):
    @functools.partial(
        pl.kernel, mesh=_sc_mesh(),
        out_type=[
            jax.ShapeDtypeStruct((NC, N, HID), jnp.float32),
            jax.ShapeDtypeStruct((NC, NE, HID), jnp.float32),
        ],
        scratch_types=[
            pltpu.VMEM((_CCHUNK,), jnp.int32),
            pltpu.VMEM((_CCHUNK, HID), jnp.float32),
            pltpu.VMEM_SHARED((N, HID), jnp.float32),
            pltpu.VMEM_SHARED((NE, HID), jnp.float32),
        ],
    )
    def k(nodes_hbm, edges_hbm, z_hbm, ones_hbm, d_out, bd_out,
          idx_v, ones_v, d_sh, bd_sh):
        c = lax.axis_index("c")
        s = lax.axis_index("s")
        stripe = s * _STRIPE
        pltpu.sync_copy(z_hbm.at[pl.ds(stripe, _STRIPE)],
                        d_sh.at[pl.ds(stripe, _STRIPE)])
        pltpu.sync_copy(z_hbm.at[pl.ds(stripe, _STRIPE)],
                        bd_sh.at[pl.ds(stripe, _STRIPE)])
        pltpu.sync_copy(ones_hbm, ones_v)
        plsc.subcore_barrier()
        base = c * (EDGES // NC) + s * _EDGE_PER_TILE
        for j in range(_EDGE_PER_TILE // _CCHUNK):
            pltpu.sync_copy(nodes_hbm.at[pl.ds(base + j * _CCHUNK, _CCHUNK)],
                            idx_v)
            pltpu.sync_copy(ones_v, d_sh.at[idx_v], add=True)
            pltpu.sync_copy(edges_hbm.at[pl.ds(base + j * _CCHUNK, _CCHUNK)],
                            idx_v)
            pltpu.sync_copy(ones_v, bd_sh.at[idx_v], add=True)
        plsc.subcore_barrier()
        pltpu.sync_copy(d_sh.at[pl.ds(stripe, _STRIPE)],
                        d_out.at[c].at[pl.ds(stripe, _STRIPE)])
        pltpu.sync_copy(bd_sh.at[pl.ds(stripe, _STRIPE)],
                        bd_out.at[c].at[pl.ds(stripe, _STRIPE)])

    return k(nodes, edges, zeros128, ones128)


def _sc_segsum(values, gather_idx, scatter_idx, zeros128):
    @functools.partial(
        pl.kernel, mesh=_sc_mesh(),
        out_type=jax.ShapeDtypeStruct((NC, N, HID), jnp.float32),
        scratch_types=[
            pltpu.VMEM((_CHUNK,), jnp.int32),
            pltpu.VMEM((_CHUNK,), jnp.int32),
            pltpu.VMEM((_CHUNK, HID), jnp.float32),
            pltpu.VMEM_SHARED((N, HID), jnp.float32),
            pltpu.SemaphoreType.DMA,
        ],
    )
    def k(val_hbm, gi_hbm, si_hbm, z_hbm, o_hbm, gi_v, si_v, rows_v, acc_sh,
          sem):
        c = lax.axis_index("c")
        s = lax.axis_index("s")
        stripe = s * _STRIPE
        pltpu.sync_copy(z_hbm.at[pl.ds(stripe, _STRIPE)],
                        acc_sh.at[pl.ds(stripe, _STRIPE)])
        plsc.subcore_barrier()
        base = c * (EDGES // NC) + s * _EDGE_PER_TILE
        for j in range(_EDGE_CHUNKS):
            off = base + j * _CHUNK
            pltpu.sync_copy(gi_hbm.at[pl.ds(off, _CHUNK)], gi_v)
            pltpu.sync_copy(si_hbm.at[pl.ds(off, _CHUNK)], si_v)
            pltpu.async_copy(val_hbm.at[gi_v], rows_v, sem).wait()
            pltpu.sync_copy(rows_v, acc_sh.at[si_v], add=True)
        plsc.subcore_barrier()
        pltpu.sync_copy(acc_sh.at[pl.ds(stripe, _STRIPE)],
                        o_hbm.at[c].at[pl.ds(stripe, _STRIPE)])

    return k(values, gather_idx, scatter_idx, zeros128)


def kernel(embs1, embs2, m_embs, edge_index, W_hg, b_hg, bn_gamma, bn_beta,
           attn_in_w, attn_in_b, attn_out_w, attn_out_b,
           mha_in_w, mha_in_b, mha_out_w, mha_out_b):
    x = jnp.concatenate([embs1, embs2], axis=0)
    nodes = edge_index[0]
    edges = edge_index[1]
    he = jax.random.normal(jax.random.key(1), (N, HID), dtype=jnp.float32)

    zeros128 = jnp.zeros((N, HID), jnp.float32)
    ones128 = jnp.ones((_CCHUNK, HID), jnp.float32)

    d_part, bd_part = _sc_counts(nodes, edges, zeros128, ones128)

    he_out = _tc_mha(he, x, attn_in_w, attn_in_b.reshape(1, -1),
                     attn_out_w, attn_out_b.reshape(1, -1))
    xw = _tc_knn_mm(x, he_out, W_hg)

    ef_part = _sc_segsum(xw, nodes, edges, zeros128)
    ef = _tc_combine_ef(ef_part, bd_part)
    out_part = _sc_segsum(ef, edges, nodes, zeros128)

    return _tc_final(out_part, d_part, b_hg.reshape(1, -1),
                     bn_gamma.reshape(1, -1), bn_beta.reshape(1, -1),
                     m_embs, mha_in_w, mha_in_b.reshape(1, -1),
                     mha_out_w, mha_out_b.reshape(1, -1))
